# Initial kernel scaffold; baseline (speedup 1.0000x reference)
import jax
import jax.numpy as jnp
from jax.experimental import pallas as pl


def kernel(edges, init_state, deg_W1, deg_b1, deg_W2, deg_b2, msg_W1, msg_b1, msg_W2, msg_b2, lstm_Wih, lstm_Whh, lstm_bih, lstm_bhh, ln_g, ln_b, W_out):
    raise NotImplementedError("write your pallas kernel here")


# speedup vs baseline: 2.0928x; 2.0928x over previous
import functools

import jax
import jax.numpy as jnp
from jax import lax
from jax.experimental import pallas as pl
from jax.experimental.pallas import tpu as pltpu
from jax.experimental.pallas import tpu_sc as plsc

K = 128
NB = 10
N = 10000
E = 320000
EPS = 1e-5

NTILES = 16
NR = 10240
JUNK = N
CHUNK = 128
EP = 641024
CPT = EP // (NTILES * CHUNK)
DCH = 64
DCPT = (EP // 2) // (NTILES * DCH)
BLK = 1024
GRID = NR // BLK

_MESH = plsc.VectorSubcoreMesh(core_axis_name="c", subcore_axis_name="s")


def _deg_body(dsts, out, dst_b, val_b, deg_sh):
    c = lax.axis_index("c")
    w = lax.axis_index("s")
    zero = jnp.zeros((16,), jnp.float32)
    one = jnp.ones((16,), jnp.float32)

    def _fill(e, carry):
        val_b[e, pl.ds(0, 16)] = zero
        return carry

    lax.fori_loop(0, DCH, _fill, 0)
    for b in range(10):
        pltpu.sync_copy(val_b, deg_sh.at[pl.ds(w * 640 + b * DCH, DCH)])

    def _fill1(e, carry):
        val_b[e, pl.ds(0, 16)] = one
        return carry

    lax.fori_loop(0, DCH, _fill1, 0)
    plsc.subcore_barrier()

    base0 = c * (EP // 2) + w * (DCPT * DCH)

    def _chunk(i, carry):
        pltpu.sync_copy(dsts.at[pl.ds(base0 + i * DCH, DCH)], dst_b)
        pltpu.sync_copy(val_b, deg_sh.at[dst_b], add=True)
        return carry

    lax.fori_loop(0, DCPT, _chunk, 0)
    plsc.subcore_barrier()
    pltpu.sync_copy(deg_sh.at[pl.ds(w * 640, 640)], out.at[c, pl.ds(w * 640, 640)])


_deg_call = pl.kernel(
    _deg_body,
    out_type=jax.ShapeDtypeStruct((2, NR, 16), jnp.float32),
    mesh=_MESH,
    scratch_types=[
        pltpu.VMEM((DCH,), jnp.int32),
        pltpu.VMEM((DCH, 16), jnp.float32),
        pltpu.VMEM_SHARED((NR, 16), jnp.float32),
    ],
)


def _edge_body(dsts, srcs, ptbl, qtbl, out, dst_b, src_b, iq_b, p_b, q_b, agg,
               sem1, sem2):
    c = lax.axis_index("c")
    w = lax.axis_index("s")
    zero = jnp.zeros((16,), jnp.float32)

    def _zb(e, carry):
        for j in range(8):
            p_b[e, pl.ds(j * 16, 16)] = zero
        return carry

    lax.fori_loop(0, CHUNK, _zb, 0)
    for b in range(5):
        pltpu.sync_copy(p_b, agg.at[pl.ds(w * 640 + b * CHUNK, CHUNK)])
    plsc.subcore_barrier()

    off = c * NR
    base0 = w * (CPT * CHUNK)

    def _chunk(i, carry):
        base = base0 + i * CHUNK
        pltpu.sync_copy(dsts.at[pl.ds(base, CHUNK)], dst_b)
        pltpu.sync_copy(srcs.at[pl.ds(base, CHUNK)], src_b)
        for j in range(8):
            sl = pl.ds(j * 16, 16)
            src_b[sl] = src_b[sl] + off
            iq_b[sl] = dst_b[sl] + off
        cp = pltpu.async_copy(ptbl.at[src_b], p_b, sem1)
        cq = pltpu.async_copy(qtbl.at[iq_b], q_b, sem2)
        cp.wait()
        cq.wait()

        def _relu(e, cc):
            for j in range(8):
                sl = pl.ds(j * 16, 16)
                p_b[e, sl] = jnp.maximum(p_b[e, sl] + q_b[e, sl], 0.0)
            return cc

        lax.fori_loop(0, CHUNK, _relu, 0)
        pltpu.sync_copy(p_b, agg.at[dst_b], add=True)
        return carry

    lax.fori_loop(0, CPT, _chunk, 0)
    plsc.subcore_barrier()
    pltpu.sync_copy(agg.at[pl.ds(w * 640, 640)], out.at[c, pl.ds(w * 640, 640)])


_edge_call = pl.kernel(
    _edge_body,
    out_type=jax.ShapeDtypeStruct((2, NR, K), jnp.float32),
    mesh=_MESH,
    scratch_types=[
        pltpu.VMEM((CHUNK,), jnp.int32),
        pltpu.VMEM((CHUNK,), jnp.int32),
        pltpu.VMEM((CHUNK,), jnp.int32),
        pltpu.VMEM((CHUNK, K), jnp.float32),
        pltpu.VMEM((CHUNK, K), jnp.float32),
        pltpu.VMEM_SHARED((NR, K), jnp.float32),
        pltpu.SemaphoreType.DMA,
        pltpu.SemaphoreType.DMA,
    ],
)


def _dot(a, b):
    return jnp.dot(a, b, preferred_element_type=jnp.float32)


def _node0_body(degp, istate, w1row, b1row, W2T, b2row, Wpa, Wpb, Wqa, Wqb,
                b1a, b1b, s_o, dr_o, p_o, q_o):
    dr = degp[0] + degp[1]
    dr_o[...] = dr
    dc = jnp.maximum(dr[:, :1], 1.0)
    x = jnp.maximum(dc * w1row[...] + b1row[...], 0.0)
    s = istate[...] + _dot(x, W2T[...]) + b2row[...]
    s_o[...] = s
    p_o[0] = _dot(s, Wpa[...])
    p_o[1] = _dot(s, Wpb[...])
    q_o[0] = _dot(s, Wqa[...]) + b1a[...]
    q_o[1] = _dot(s, Wqb[...]) + b1b[...]


def _node1_body(agg, dr, s_in, W2aT, W2bT, b2row, WihT, bsum, lng, lnb,
                Wpa, Wpb, Wqa, Wqb, b1a, b1b, s_o, h_o, p_o, q_o):
    s = s_in[...]
    msg = _dot(agg[0], W2aT[...]) + _dot(agg[1], W2bT[...]) \
        + dr[:, :1] * b2row[...]
    dc = jnp.maximum(dr[:, :1], 1.0)
    r = msg / dc
    gates = _dot(r, WihT[...]) + bsum[...]
    i_g = jax.nn.sigmoid(gates[:, :K])
    f_g = jax.nn.sigmoid(gates[:, K:2 * K])
    g_g = jnp.tanh(gates[:, 2 * K:3 * K])
    o_g = jax.nn.sigmoid(gates[:, 3 * K:])
    c_new = f_g * s + i_g * g_g
    h_o[...] = o_g * jnp.tanh(c_new)
    sn = s + c_new
    mu = jnp.mean(sn, axis=-1, keepdims=True)
    var = jnp.mean((sn - mu) ** 2, axis=-1, keepdims=True)
    s_new = (sn - mu) / jnp.sqrt(var + EPS) * lng[...] + lnb[...]
    s_o[...] = s_new
    p_o[0] = _dot(s_new, Wpa[...])
    p_o[1] = _dot(s_new, Wpb[...])
    q_o[0] = _dot(s_new, Wqa[...]) + b1a[...]
    q_o[1] = _dot(s_new, Wqb[...]) + b1b[...]


def _node2_body(agg, dr, s_in, h_in, W2aT, W2bT, b2row, WihT, WhhT, bsum,
                lng, lnb, WoutT, out):
    s = s_in[...]
    msg = _dot(agg[0], W2aT[...]) + _dot(agg[1], W2bT[...]) \
        + dr[:, :1] * b2row[...]
    dc = jnp.maximum(dr[:, :1], 1.0)
    r = msg / dc
    gates = _dot(r, WihT[...]) + _dot(h_in[...], WhhT[...]) + bsum[...]
    i_g = jax.nn.sigmoid(gates[:, :K])
    f_g = jax.nn.sigmoid(gates[:, K:2 * K])
    g_g = jnp.tanh(gates[:, 2 * K:3 * K])
    c_new = f_g * s + i_g * g_g
    sn = s + c_new
    mu = jnp.mean(sn, axis=-1, keepdims=True)
    var = jnp.mean((sn - mu) ** 2, axis=-1, keepdims=True)
    s_new = (sn - mu) / jnp.sqrt(var + EPS) * lng[...] + lnb[...]
    out[...] = _dot(s_new, WoutT[...]) * 2.0


def _full(shape):
    nd = len(shape)
    return pl.BlockSpec(shape, lambda i, _n=nd: (0,) * _n)


_ROWS = pl.BlockSpec((BLK, K), lambda i: (i, 0))
_ROWS16 = pl.BlockSpec((BLK, 16), lambda i: (i, 0))
_ROWS2 = pl.BlockSpec((2, BLK, K), lambda i: (0, i, 0))
_SDS_ROWS = jax.ShapeDtypeStruct((NR, K), jnp.float32)
_SDS_ROWS16 = jax.ShapeDtypeStruct((NR, 16), jnp.float32)
_SDS_ROWS2 = jax.ShapeDtypeStruct((2, NR, K), jnp.float32)

_node0_call = pl.pallas_call(
    _node0_body,
    grid=(GRID,),
    in_specs=[
        pl.BlockSpec((2, BLK, 16), lambda i: (0, i, 0)),
        _full((1, K)), _full((1, K)), _full((1, K)), _full((K, K)),
        _full((1, K)), _full((K, K)), _full((K, K)), _full((K, K)),
        _full((K, K)), _full((1, K)), _full((1, K)),
    ],
    out_specs=[_ROWS, _ROWS16, _ROWS2, _ROWS2],
    out_shape=[_SDS_ROWS, _SDS_ROWS16, _SDS_ROWS2, _SDS_ROWS2],
)

_node1_call = pl.pallas_call(
    _node1_body,
    grid=(GRID,),
    in_specs=[
        _ROWS2, _ROWS16, _ROWS,
        _full((K, K)), _full((K, K)), _full((1, K)),
        _full((K, 4 * K)), _full((1, 4 * K)), _full((1, K)), _full((1, K)),
        _full((K, K)), _full((K, K)), _full((K, K)), _full((K, K)),
        _full((1, K)), _full((1, K)),
    ],
    out_specs=[_ROWS, _ROWS, _ROWS2, _ROWS2],
    out_shape=[_SDS_ROWS, _SDS_ROWS, _SDS_ROWS2, _SDS_ROWS2],
)

_node2_call = pl.pallas_call(
    _node2_body,
    grid=(GRID,),
    in_specs=[
        _ROWS2, _ROWS16, _ROWS, _ROWS,
        _full((K, K)), _full((K, K)), _full((1, K)),
        _full((K, 4 * K)), _full((K, 4 * K)), _full((1, 4 * K)),
        _full((1, K)), _full((1, K)), _full((K, K)),
    ],
    out_specs=[_ROWS],
    out_shape=[_SDS_ROWS],
)


def kernel(edges, init_state, deg_W1, deg_b1, deg_W2, deg_b2, msg_W1, msg_b1,
           msg_W2, msg_b2, lstm_Wih, lstm_Whh, lstm_bih, lstm_bhh, ln_g, ln_b,
           W_out):
    u = edges[0]
    v = edges[1]
    npad = EP - 2 * E
    dsts = jnp.concatenate([v, u, jnp.full((npad,), JUNK, jnp.int32)])
    srcs = jnp.concatenate([u, v, jnp.zeros((npad,), jnp.int32)])

    istate = init_state[None, :]
    w1row = deg_W1.T
    b1row = deg_b1[None, :]
    W2T = deg_W2.T
    b2row = deg_b2[None, :]
    Wpa = msg_W1[:K, :K].T
    Wpb = msg_W1[K:, :K].T
    Wqa = msg_W1[:K, K:].T
    Wqb = msg_W1[K:, K:].T
    b1a = msg_b1[None, :K]
    b1b = msg_b1[None, K:]
    W2aT = msg_W2[:, :K].T
    W2bT = msg_W2[:, K:].T
    mb2row = msg_b2[None, :]
    WihT = lstm_Wih.T
    WhhT = lstm_Whh.T
    bsum = (lstm_bih + lstm_bhh)[None, :]
    lng = ln_g[None, :]
    lnb = ln_b[None, :]
    WoutT = jnp.pad(W_out.T, ((0, 0), (0, K - NB)))

    degp = _deg_call(dsts)
    s, dr, p_o, q_o = _node0_call(degp, istate, w1row, b1row, W2T, b2row,
                                  Wpa, Wpb, Wqa, Wqb, b1a, b1b)
    agg = _edge_call(dsts, srcs, p_o.reshape(2 * NR, K), q_o.reshape(2 * NR, K))
    s, h, p_o, q_o = _node1_call(agg, dr, s, W2aT, W2bT, mb2row, WihT, bsum,
                                 lng, lnb, Wpa, Wpb, Wqa, Wqb, b1a, b1b)
    agg = _edge_call(dsts, srcs, p_o.reshape(2 * NR, K), q_o.reshape(2 * NR, K))
    (out,) = _node2_call(agg, dr, s, h, W2aT, W2bT, mb2row, WihT, WhhT, bsum,
                         lng, lnb, WoutT)
    return out[:N, :NB]
